# Initial kernel scaffold; baseline (speedup 1.0000x reference)
import jax
import jax.numpy as jnp
from jax.experimental import pallas as pl


def kernel(x, tok_table, pos_table, gamma, beta):
    raise NotImplementedError("write your pallas kernel here")


# speedup vs baseline: 1.8643x; 1.8643x over previous
import functools

import jax
import jax.numpy as jnp
from jax import lax
from jax.experimental import pallas as pl
from jax.experimental.pallas import tpu as pltpu
from jax.experimental.pallas import tpu_sc as plsc

BATCH = 4096
SEQ = 200
D = 64
TOKENS = BATCH * SEQ

NC = 2
NS = 16
NW = NC * NS
PER_W = TOKENS // NW
CHUNK = 128
N_CHUNK = PER_W // CHUNK

_sc_mesh = plsc.VectorSubcoreMesh(core_axis_name="c", subcore_axis_name="s")


@functools.partial(
    pl.kernel,
    mesh=_sc_mesh,
    compiler_params=pltpu.CompilerParams(use_tc_tiling_on_sc=False),
    out_type=jax.ShapeDtypeStruct((TOKENS, D), jnp.float32),
    scratch_types=[
        pltpu.VMEM((CHUNK,), jnp.int32),
        pltpu.VMEM((CHUNK, D), jnp.float32),
        pltpu.SemaphoreType.DMA,
    ],
)
def _sc_gather(idx_hbm, table_hbm, out_hbm, idx_v, rows_v, sem):
    wid = lax.axis_index("s") * NC + lax.axis_index("c")
    base = wid * PER_W

    def body(k, carry):
        start = base + k * CHUNK
        pltpu.sync_copy(idx_hbm.at[pl.ds(start, CHUNK)], idx_v)
        pltpu.async_copy(table_hbm.at[idx_v], rows_v, sem).wait()
        pltpu.sync_copy(rows_v, out_hbm.at[pl.ds(start, CHUNK)])
        return carry

    lax.fori_loop(0, N_CHUNK, body, 0)


ROWS_BLK = 1600


def _ln_body(emb_ref, pos_ref, gamma_ref, beta_ref, out_ref):
    e = emb_ref[...] + pos_ref[...]
    m = jnp.mean(e, axis=1, keepdims=True)
    c = e - m
    v = jnp.mean(c * c, axis=1, keepdims=True)
    out_ref[...] = c * lax.rsqrt(v + 1e-5) * gamma_ref[...] + beta_ref[...]


_ln_call = pl.pallas_call(
    _ln_body,
    grid=(TOKENS // ROWS_BLK,),
    in_specs=[
        pl.BlockSpec((ROWS_BLK, D), lambda i: (i, 0)),
        pl.BlockSpec((ROWS_BLK, D), lambda i: (0, 0)),
        pl.BlockSpec((1, D), lambda i: (0, 0)),
        pl.BlockSpec((1, D), lambda i: (0, 0)),
    ],
    out_specs=pl.BlockSpec((ROWS_BLK, D), lambda i: (i, 0)),
    out_shape=jax.ShapeDtypeStruct((TOKENS, D), jnp.float32),
)


def kernel(x, tok_table, pos_table, gamma, beta):
    idx = x.reshape(-1).astype(jnp.int32)
    emb = _sc_gather(idx, tok_table)
    pos_tiled = jnp.tile(pos_table[:SEQ], (ROWS_BLK // SEQ, 1))
    out = _ln_call(emb, pos_tiled, gamma.reshape(1, D), beta.reshape(1, D))
    return out.reshape(BATCH, SEQ, D)
